# baseline (speedup 1.0000x reference)
import jax
import jax.numpy as jnp
from jax import lax
from jax.experimental import pallas as pl
from jax.experimental.pallas import tpu as pltpu
from jax.experimental.pallas import tpu_sc as plsc

MAX_DEGREE = 256
NODE_DIM = 128
CPE_DIM = 64
N_NODES = 10000
N_PAD = 10240
N_EDGES = 320000

NC = 2
NS = 16
L = 16

EDGES_PER_TILE = N_EDGES // NS
NODES_PER_SC = N_PAD // NC
CHUNK = 320
GCH = 80
SB = 10


def _body(edge_hbm, z_hbm, cpe_hbm,
          edge_v, hist_v, hist16_v, shared16, red16_v, deg_v, cpe_v,
          sem_r, sem_g):
  c = lax.axis_index("c")
  s = lax.axis_index("s")
  gbase = c * NODES_PER_SC + s * CHUNK

  edge_load = pltpu.async_copy(
      edge_hbm.at[pl.ds(s * EDGES_PER_TILE, EDGES_PER_TILE)], edge_v, sem_g)

  zeros16 = jnp.zeros((L,), jnp.int32)

  def zero_body(i, _):
    hist_v[pl.ds(i * L, L)] = zeros16
    return 0
  lax.fori_loop(0, N_PAD // L, zero_body, 0, unroll=8)

  edge_load.wait()
  ones16 = jnp.ones((L,), jnp.int32)

  def scat_body(i, _):
    evs = [edge_v[pl.ds((i * SB + b) * L, L)] for b in range(SB)]
    for ev in evs:
      plsc.addupdate_scatter(hist_v, [ev], ones16)
    return 0
  lax.fori_loop(0, EDGES_PER_TILE // (L * SB), scat_body, 0)

  def pack_body(i, _):
    a = hist_v[pl.ds(2 * L * i, L)]
    b = hist_v[pl.ds(2 * L * i + L, L)]
    hist16_v[pl.ds(2 * L * i, 2 * L)] = plsc.pack(
        a, b, format=plsc.PackFormat.INTERLEAVED)
    return 0
  lax.fori_loop(0, N_PAD // (2 * L), pack_body, 0, unroll=4)

  pltpu.sync_copy(hist16_v, shared16.at[pl.ds(s * N_PAD, N_PAD)])
  plsc.subcore_barrier()

  red_reads = []
  for r in range(NS):
    red_reads.append(pltpu.async_copy(
        shared16.at[pl.ds(r * N_PAD + gbase, CHUNK)],
        red16_v.at[pl.ds(r * CHUNK, CHUNK)], sem_r))
  for cp in red_reads:
    cp.wait()

  maxd = jnp.full((L,), MAX_DEGREE, jnp.int32)
  for g in range(CHUNK // (2 * L)):
    acc_a = jnp.zeros((L,), jnp.int32)
    acc_b = jnp.zeros((L,), jnp.int32)
    for r in range(NS):
      v = red16_v[pl.ds(r * CHUNK + 2 * L * g, 2 * L)]
      a, b = plsc.unpack(v, format=plsc.PackFormat.INTERLEAVED,
                         preferred_element_type=jnp.int32)
      acc_a = acc_a + a
      acc_b = acc_b + b
    pa = 2 * L * g
    pb = 2 * L * g + L
    deg_v[pa // GCH, pl.ds(pa % GCH, L)] = jnp.minimum(acc_a, maxd)
    deg_v[pb // GCH, pl.ds(pb % GCH, L)] = jnp.minimum(acc_b, maxd)

  gathers = []
  for k in range(CHUNK // GCH):
    gathers.append(pltpu.async_copy(
        z_hbm.at[deg_v.at[k]], cpe_v.at[pl.ds(k * GCH, GCH)], sem_g))
  for cp in gathers:
    cp.wait()

  pltpu.sync_copy(cpe_v, cpe_hbm.at[pl.ds(gbase, CHUNK), :])


@jax.jit
def kernel(x, edge_index, z):
  edge_flat = edge_index.astype(jnp.int32).reshape(-1)
  mesh = plsc.VectorSubcoreMesh(core_axis_name="c", subcore_axis_name="s",
                                num_cores=NC, num_subcores=NS)
  f = pl.kernel(
      _body,
      out_type=jax.ShapeDtypeStruct((N_PAD, CPE_DIM), jnp.float32),
      mesh=mesh,
      compiler_params=pltpu.CompilerParams(needs_layout_passes=False,
                                           use_tc_tiling_on_sc=False),
      scratch_types=[
          pltpu.VMEM((EDGES_PER_TILE,), jnp.int32),
          pltpu.VMEM((N_PAD,), jnp.int32),
          pltpu.VMEM((N_PAD,), jnp.int16),
          pltpu.VMEM_SHARED((NS * N_PAD,), jnp.int16),
          pltpu.VMEM((NS * CHUNK,), jnp.int16),
          pltpu.VMEM((CHUNK // GCH, GCH), jnp.int32),
          pltpu.VMEM((CHUNK, CPE_DIM), jnp.float32),
          pltpu.SemaphoreType.DMA,
          pltpu.SemaphoreType.DMA,
      ],
  )
  cpe = f(edge_flat, z)
  return jnp.concatenate((x, cpe[:N_NODES]), axis=1)


# speedup vs baseline: 1.0598x; 1.0598x over previous
import jax
import jax.numpy as jnp
from jax import lax
from jax.experimental import pallas as pl
from jax.experimental.pallas import tpu as pltpu
from jax.experimental.pallas import tpu_sc as plsc

MAX_DEGREE = 256
NODE_DIM = 128
CPE_DIM = 64
N_NODES = 10000
N_EDGES = 320000

NC = 2
NS = 16
L = 16

EDGES_PER_TILE = N_EDGES // NS
NODES_PER_SC = N_NODES // NC
CHUNK = 320
BSTEP = 312
GCH = 80
SB = 25


def _body(edge_hbm, z_hbm, cpe_hbm,
          edge_v, hist_v, shared_hist, red_v, deg_v, cpe_v, sem_r, sem_g):
  c = lax.axis_index("c")
  s = lax.axis_index("s")
  gbase = c * NODES_PER_SC + s * BSTEP

  edge_load = pltpu.async_copy(
      edge_hbm.at[pl.ds(s * EDGES_PER_TILE, EDGES_PER_TILE)], edge_v, sem_g)

  zeros16 = jnp.zeros((L,), jnp.int32)

  def zero_body(i, _):
    hist_v[pl.ds(i * L, L)] = zeros16
    return 0
  lax.fori_loop(0, N_NODES // L, zero_body, 0, unroll=8)

  edge_load.wait()
  ones16 = jnp.ones((L,), jnp.int32)

  def scat_body(i, _):
    evs = [edge_v[pl.ds((i * SB + b) * L, L)] for b in range(SB)]
    for ev in evs:
      plsc.addupdate_scatter(hist_v, [ev], ones16)
    return 0
  lax.fori_loop(0, EDGES_PER_TILE // (L * SB), scat_body, 0)

  pltpu.sync_copy(hist_v, shared_hist.at[pl.ds(s * N_NODES, N_NODES)])
  plsc.subcore_barrier()

  red_reads = []
  for r in range(NS):
    red_reads.append(pltpu.async_copy(
        shared_hist.at[pl.ds(r * N_NODES + gbase, CHUNK)],
        red_v.at[pl.ds(r * CHUNK, CHUNK)], sem_r))
  for cp in red_reads:
    cp.wait()

  maxd = jnp.full((L,), MAX_DEGREE, jnp.int32)
  for k in range(CHUNK // GCH):
    for m in range(GCH // L):
      off = k * GCH + m * L
      acc = red_v[pl.ds(off, L)]
      for r in range(1, NS):
        acc = acc + red_v[pl.ds(r * CHUNK + off, L)]
      deg_v[k, pl.ds(m * L, L)] = jnp.minimum(acc, maxd)

  gathers = []
  for k in range(CHUNK // GCH):
    gathers.append(pltpu.async_copy(
        z_hbm.at[deg_v.at[k]], cpe_v.at[pl.ds(k * GCH, GCH)], sem_g))
  for cp in gathers:
    cp.wait()

  pltpu.sync_copy(cpe_v, cpe_hbm.at[pl.ds(gbase, CHUNK), :])


@jax.jit
def kernel(x, edge_index, z):
  edge_flat = edge_index.astype(jnp.int32).reshape(-1)
  mesh = plsc.VectorSubcoreMesh(core_axis_name="c", subcore_axis_name="s",
                                num_cores=NC, num_subcores=NS)
  f = pl.kernel(
      _body,
      out_type=jax.ShapeDtypeStruct((N_NODES, CPE_DIM), jnp.float32),
      mesh=mesh,
      compiler_params=pltpu.CompilerParams(needs_layout_passes=False,
                                           use_tc_tiling_on_sc=False),
      scratch_types=[
          pltpu.VMEM((EDGES_PER_TILE,), jnp.int32),
          pltpu.VMEM((N_NODES,), jnp.int32),
          pltpu.VMEM_SHARED((NS * N_NODES,), jnp.int32),
          pltpu.VMEM((NS * CHUNK,), jnp.int32),
          pltpu.VMEM((CHUNK // GCH, GCH), jnp.int32),
          pltpu.VMEM((CHUNK, CPE_DIM), jnp.float32),
          pltpu.SemaphoreType.DMA,
          pltpu.SemaphoreType.DMA,
      ],
  )
  cpe = f(edge_flat, z)
  return jnp.concatenate((x, cpe), axis=1)
